# initial kernel scaffold (unmeasured)
import jax
import jax.numpy as jnp
from jax import lax
from jax.experimental import pallas as pl
from jax.experimental.pallas import tpu as pltpu

N_DEV = 8
S = 1024
H = 8
DH = 128
D = H * DH
BLK = 64
SCALE = 0.08838834764831843
BF = jnp.bfloat16
F32 = jnp.float32


def kernel(x, Wq, K_ext, V_ext, Wo):
    x2 = x.reshape(S, D)
    k2 = K_ext.reshape(S, D)
    v2 = V_ext.reshape(S, D)

    def body(x_ref, wq_ref, k_ref, v_ref, wo_ref, out_ref,
             kv_buf, q_scr, acc, m_scr, l_scr, send_sems, recv_sems):
        my = lax.axis_index("i")
        left = lax.rem(my + N_DEV - 1, N_DEV)
        right = lax.rem(my + 1, N_DEV)

        barrier_sem = pltpu.get_barrier_semaphore()
        for nbr in (left, right):
            pl.semaphore_signal(barrier_sem, inc=1, device_id=(nbr,),
                                device_id_type=pl.DeviceIdType.MESH)
        pl.semaphore_wait(barrier_sem, 2)

        q = lax.dot_general(
            x_ref[...].astype(BF), wq_ref[...].astype(BF),
            (((1,), (0,)), ((), ())), preferred_element_type=F32)
        q_scr[...] = (q * SCALE).astype(BF)

        kv_buf[0, :, 0:D] = k_ref[...].astype(BF)
        kv_buf[0, :, D:2 * D] = v_ref[...].astype(BF)

        m_scr[...] = jnp.full((H, S, 1), -1e30, F32)
        l_scr[...] = jnp.zeros((H, S, 1), F32)
        acc[...] = jnp.zeros((S, D), F32)

        def process(slot, j):
            @pl.when(j <= my)
            def _():
                rows = lax.broadcasted_iota(jnp.int32, (S, S), 0)
                cols = lax.broadcasted_iota(jnp.int32, (S, S), 1)
                qb = my * (S // BLK) + rows // BLK
                kb = j * (S // BLK) + cols // BLK
                bias = jnp.where(kb <= qb, 0.0, -1e9).astype(F32)

                def head(h, _):
                    qh = q_scr[:, pl.ds(h * DH, DH)]
                    kh = kv_buf[slot, :, pl.ds(h * DH, DH)]
                    vh = kv_buf[slot, :, pl.ds(D + h * DH, DH)]
                    s = lax.dot_general(qh, kh, (((1,), (1,)), ((), ())),
                                        preferred_element_type=F32)
                    s = s + bias
                    m_old = m_scr[h]
                    m_new = jnp.maximum(m_old, jnp.max(s, axis=1, keepdims=True))
                    p = jnp.exp(s - m_new)
                    alpha = jnp.exp(m_old - m_new)
                    l_scr[h] = l_scr[h] * alpha + jnp.sum(p, axis=1, keepdims=True)
                    pv = lax.dot_general(p.astype(BF), vh,
                                         (((1,), (0,)), ((), ())),
                                         preferred_element_type=F32)
                    acc[:, pl.ds(h * DH, DH)] = (
                        acc[:, pl.ds(h * DH, DH)] * alpha + pv)
                    m_scr[h] = m_new
                    return 0

                lax.fori_loop(0, H, head, 0)

        for hop in range(1, N_DEV):
            send_slot = (hop - 1) % 2
            recv_slot = hop % 2
            rdma = pltpu.make_async_remote_copy(
                src_ref=kv_buf.at[send_slot],
                dst_ref=kv_buf.at[recv_slot],
                send_sem=send_sems.at[hop - 1],
                recv_sem=recv_sems.at[hop - 1],
                device_id=(right,),
                device_id_type=pl.DeviceIdType.MESH)
            rdma.start()
            process(send_slot, lax.rem(my + N_DEV - (hop - 1), N_DEV))
            rdma.wait()
        process((N_DEV - 1) % 2, lax.rem(my + 1, N_DEV))

        def fin(h, _):
            acc[:, pl.ds(h * DH, DH)] = acc[:, pl.ds(h * DH, DH)] / l_scr[h]
            return 0
        lax.fori_loop(0, H, fin, 0)

        out_ref[...] = lax.dot_general(
            acc[...].astype(BF), wo_ref[...].astype(BF),
            (((1,), (0,)), ((), ())), preferred_element_type=F32)

    out2 = pl.pallas_call(
        body,
        out_shape=jax.ShapeDtypeStruct((S, D), F32),
        in_specs=[pl.BlockSpec(memory_space=pltpu.VMEM)] * 5,
        out_specs=pl.BlockSpec(memory_space=pltpu.VMEM),
        scratch_shapes=[
            pltpu.VMEM((2, S, 2 * D), BF),
            pltpu.VMEM((S, D), BF),
            pltpu.VMEM((S, D), F32),
            pltpu.VMEM((H, S, 1), F32),
            pltpu.VMEM((H, S, 1), F32),
            pltpu.SemaphoreType.DMA((N_DEV - 1,)),
            pltpu.SemaphoreType.DMA((N_DEV - 1,)),
        ],
        compiler_params=pltpu.CompilerParams(collective_id=0),
    )(x2, Wq, k2, v2, Wo)
    return out2.reshape(1, S, D)


# baseline (device time: 349020 ns/iter reference)
import jax
import jax.numpy as jnp
from jax import lax
from jax.experimental import pallas as pl
from jax.experimental.pallas import tpu as pltpu

N_DEV = 8
S = 1024
H = 8
DH = 128
D = H * DH
BLK = 64
SCALE = 0.08838834764831843
BF = jnp.bfloat16
F32 = jnp.float32


def kernel(x, Wq, K_ext, V_ext, Wo):
    x2 = x.reshape(S, D)
    k2 = K_ext.reshape(S, D)
    v2 = V_ext.reshape(S, D)

    def body(x_ref, wq_ref, k_ref, v_ref, wo_ref, out_ref,
             kv_buf, q_scr, acc, m_scr, l_scr, send_sems, recv_sems):
        my = lax.axis_index("i")
        left = lax.rem(my + N_DEV - 1, N_DEV)
        right = lax.rem(my + 1, N_DEV)

        barrier_sem = pltpu.get_barrier_semaphore()
        for nbr in (left, right):
            pl.semaphore_signal(barrier_sem, inc=1, device_id=(nbr,),
                                device_id_type=pl.DeviceIdType.MESH)
        pl.semaphore_wait(barrier_sem, 2)

        q = lax.dot_general(
            x_ref[...].astype(BF), wq_ref[...].astype(BF),
            (((1,), (0,)), ((), ())), preferred_element_type=F32)
        q_scr[...] = (q * SCALE).astype(BF)

        kv_buf[0, :, 0:D] = k_ref[...].astype(BF)
        kv_buf[0, :, D:2 * D] = v_ref[...].astype(BF)

        m_scr[...] = jnp.full((H, S, 1), -1e30, F32)
        l_scr[...] = jnp.zeros((H, S, 1), F32)
        acc[...] = jnp.zeros((S, D), F32)

        QT = 256

        def process(slot, j):
            @pl.when(j <= my)
            def _():
                def head(h, _):
                    kh = kv_buf[slot, :, pl.ds(h * DH, DH)]
                    vh = kv_buf[slot, :, pl.ds(D + h * DH, DH)]

                    def tile(t, _):
                        r0 = t * QT
                        rows = lax.broadcasted_iota(jnp.int32, (QT, S), 0) + r0
                        cols = lax.broadcasted_iota(jnp.int32, (QT, S), 1)
                        qb = my * (S // BLK) + rows // BLK
                        kb = j * (S // BLK) + cols // BLK
                        bias = jnp.where(kb <= qb, 0.0, -1e9).astype(F32)
                        qh = q_scr[pl.ds(r0, QT), pl.ds(h * DH, DH)]
                        s = lax.dot_general(qh, kh, (((1,), (1,)), ((), ())),
                                            preferred_element_type=F32)
                        s = s + bias
                        m_old = m_scr[h, pl.ds(r0, QT)]
                        m_new = jnp.maximum(
                            m_old, jnp.max(s, axis=1, keepdims=True))
                        p = jnp.exp(s - m_new)
                        alpha = jnp.exp(m_old - m_new)
                        l_scr[h, pl.ds(r0, QT)] = (
                            l_scr[h, pl.ds(r0, QT)] * alpha
                            + jnp.sum(p, axis=1, keepdims=True))
                        pv = lax.dot_general(p.astype(BF), vh,
                                             (((1,), (0,)), ((), ())),
                                             preferred_element_type=F32)
                        acc[pl.ds(r0, QT), pl.ds(h * DH, DH)] = (
                            acc[pl.ds(r0, QT), pl.ds(h * DH, DH)] * alpha + pv)
                        m_scr[h, pl.ds(r0, QT)] = m_new
                        return 0

                    lax.fori_loop(0, S // QT, tile, 0)
                    return 0

                lax.fori_loop(0, H, head, 0)

        for hop in range(1, N_DEV):
            send_slot = (hop - 1) % 2
            recv_slot = hop % 2
            rdma = pltpu.make_async_remote_copy(
                src_ref=kv_buf.at[send_slot],
                dst_ref=kv_buf.at[recv_slot],
                send_sem=send_sems.at[hop - 1],
                recv_sem=recv_sems.at[hop - 1],
                device_id=(right,),
                device_id_type=pl.DeviceIdType.MESH)
            rdma.start()
            process(send_slot, lax.rem(my + N_DEV - (hop - 1), N_DEV))
            rdma.wait()
        process((N_DEV - 1) % 2, lax.rem(my + 1, N_DEV))

        def fin(h, _):
            acc[:, pl.ds(h * DH, DH)] = acc[:, pl.ds(h * DH, DH)] / l_scr[h]
            return 0
        lax.fori_loop(0, H, fin, 0)

        out_ref[...] = lax.dot_general(
            acc[...].astype(BF), wo_ref[...].astype(BF),
            (((1,), (0,)), ((), ())), preferred_element_type=F32)

    out2 = pl.pallas_call(
        body,
        out_shape=jax.ShapeDtypeStruct((S, D), F32),
        in_specs=[pl.BlockSpec(memory_space=pltpu.VMEM)] * 5,
        out_specs=pl.BlockSpec(memory_space=pltpu.VMEM),
        scratch_shapes=[
            pltpu.VMEM((2, S, 2 * D), BF),
            pltpu.VMEM((S, D), BF),
            pltpu.VMEM((S, D), F32),
            pltpu.VMEM((H, S, 1), F32),
            pltpu.VMEM((H, S, 1), F32),
            pltpu.SemaphoreType.DMA((N_DEV - 1,)),
            pltpu.SemaphoreType.DMA((N_DEV - 1,)),
        ],
        compiler_params=pltpu.CompilerParams(collective_id=0),
    )(x2, Wq, k2, v2, Wo)
    return out2.reshape(1, S, D)


# device time: 266974 ns/iter; 1.3073x vs baseline; 1.3073x over previous
import jax
import jax.numpy as jnp
from jax import lax
from jax.experimental import pallas as pl
from jax.experimental.pallas import tpu as pltpu

N_DEV = 8
S = 1024
H = 8
DH = 128
D = H * DH
BLK = 64
QT = 256
SCALE = 0.08838834764831843
BF = jnp.bfloat16
F32 = jnp.float32

DIMS = (1, 3, 4)
ROW0 = (0, 344, 688)
NROWS = (344, 344, 336)
PMAX = 344


def _masks(a):
    return (DIMS[a], DIMS[(a + 1) % 3], DIMS[(a + 2) % 3])


def _gmask(a):
    m0, m1, m2 = _masks(a)
    return (m0, m1, m0 ^ m1, m2, m2 ^ m0, m2 ^ m1, m2 ^ m0 ^ m1)


_SCHED = {0: [(None, 0)],
          1: [(None, 1), (0, 2)],
          2: [(None, 3), (0, 4), (1, 5), (2, 6)]}


def kernel(x, Wq, K_ext, V_ext, Wo):
    x2 = x.reshape(S, D)
    k2 = K_ext.reshape(S, D)
    v2 = V_ext.reshape(S, D)

    def body(x_ref, wq_ref, k_ref, v_ref, wo_ref, out_ref,
             own, pieces, q_scr, acc, m_scr, l_scr, stg,
             send_sems, recv_sems, copy_sems):
        my = lax.axis_index("i")

        barrier = pltpu.get_barrier_semaphore()
        for mask in DIMS:
            pl.semaphore_signal(barrier, inc=1, device_id=(my ^ mask,),
                                device_id_type=pl.DeviceIdType.MESH)
        pl.semaphore_wait(barrier, 3)

        ck = pltpu.make_async_copy(k_ref, acc, copy_sems.at[0])
        cv = pltpu.make_async_copy(v_ref, stg, copy_sems.at[1])
        ck.start()
        cv.start()
        ck.wait()
        cv.wait()
        own[:, 0:D] = acc[...].astype(BF)
        own[:, D:2 * D] = stg[...].astype(BF)

        def mk(a, c, src_slot, dst):
            nr = NROWS[a]
            if src_slot is None:
                src = own.at[pl.ds(ROW0[a], nr), :]
            else:
                src = pieces.at[a, src_slot, pl.ds(0, nr), :]
            return pltpu.make_async_remote_copy(
                src_ref=src,
                dst_ref=pieces.at[a, c, pl.ds(0, nr), :],
                send_sem=send_sems.at[a, c],
                recv_sem=recv_sems.at[a, c],
                device_id=(dst,),
                device_id_type=pl.DeviceIdType.MESH)

        def do_round(a, r):
            partner = my ^ _masks(a)[r]
            descs = []
            for src_slot, c in _SCHED[r]:
                d = mk(a, c, src_slot, partner)
                d.start()
                descs.append(d)
            return descs

        def wait_round_recv(a, r):
            for src_slot, c in _SCHED[r]:
                mk(a, c, src_slot, 0).wait_recv()

        all_sends = []
        for a in range(3):
            all_sends += do_round(a, 0)

        cx = pltpu.make_async_copy(x_ref, acc, copy_sems.at[0])
        cw = pltpu.make_async_copy(wq_ref, stg, copy_sems.at[1])
        cx.start()
        cw.start()
        cx.wait()
        cw.wait()
        q = lax.dot_general(acc[...].astype(BF), stg[...].astype(BF),
                            (((1,), (0,)), ((), ())),
                            preferred_element_type=F32)
        q_scr[...] = (q * SCALE).astype(BF)

        m_scr[...] = jnp.full((H, S, 1), -1e30, BF)
        l_scr[...] = jnp.zeros((H, S, 1), F32)
        acc[...] = jnp.zeros((S, D), F32)

        def flash(kv_ref, o, row0, nrows):
            @pl.when(o <= my)
            def _():
                def head(h, _):
                    kh = kv_ref[pl.ds(0, nrows) if kv_ref is not own
                                else pl.ds(row0, nrows), pl.ds(h * DH, DH)]
                    vh = kv_ref[pl.ds(0, nrows) if kv_ref is not own
                                else pl.ds(row0, nrows),
                                pl.ds(D + h * DH, DH)]

                    def tile(t, _):
                        r0q = t * QT
                        rows = lax.broadcasted_iota(
                            jnp.int32, (QT, nrows), 0) + r0q
                        cols = lax.broadcasted_iota(
                            jnp.int32, (QT, nrows), 1) + row0
                        qb = my * (S // BLK) + rows // BLK
                        kb = o * (S // BLK) + cols // BLK
                        bias = jnp.where(kb <= qb, 0.0, -1e9).astype(F32)
                        qh = q_scr[pl.ds(r0q, QT), pl.ds(h * DH, DH)]
                        s = lax.dot_general(qh, kh, (((1,), (1,)), ((), ())),
                                            preferred_element_type=F32)
                        s = s + bias
                        m_old = m_scr[h, pl.ds(r0q, QT)].astype(F32)
                        m_new = jnp.maximum(
                            m_old, jnp.max(s, axis=1, keepdims=True))
                        m_new = m_new.astype(BF).astype(F32)
                        p = jnp.exp(s - m_new)
                        alpha = jnp.exp(m_old - m_new)
                        l_scr[h, pl.ds(r0q, QT)] = (
                            l_scr[h, pl.ds(r0q, QT)] * alpha
                            + jnp.sum(p, axis=1, keepdims=True))
                        pv = lax.dot_general(p.astype(BF), vh,
                                             (((1,), (0,)), ((), ())),
                                             preferred_element_type=F32)
                        acc[pl.ds(r0q, QT), pl.ds(h * DH, DH)] = (
                            acc[pl.ds(r0q, QT), pl.ds(h * DH, DH)] * alpha
                            + pv)
                        m_scr[h, pl.ds(r0q, QT)] = m_new.astype(BF)
                        return 0

                    lax.fori_loop(0, S // QT, tile, 0)
                    return 0

                lax.fori_loop(0, H, head, 0)

        def process_piece(a, c):
            flash(pieces.at[a, c], my ^ _gmask(a)[c], ROW0[a], NROWS[a])

        flash(own, my, 0, S)

        for a in range(3):
            wait_round_recv(a, 0)
            all_sends += do_round(a, 1)
        for a in range(3):
            process_piece(a, 0)

        for a in range(3):
            wait_round_recv(a, 1)
            all_sends += do_round(a, 2)
        cwo = pltpu.make_async_copy(wo_ref, stg, copy_sems.at[0])
        cwo.start()
        for a in range(3):
            process_piece(a, 1)
            process_piece(a, 2)

        for a in range(3):
            wait_round_recv(a, 2)
        for a in range(3):
            for c in (3, 4, 5, 6):
                process_piece(a, c)

        for dsc in all_sends:
            dsc.wait_send()

        def fin(h, _):
            acc[:, pl.ds(h * DH, DH)] = acc[:, pl.ds(h * DH, DH)] / l_scr[h]
            return 0
        lax.fori_loop(0, H, fin, 0)

        cwo.wait()
        out_ref[...] = lax.dot_general(
            acc[...].astype(BF), stg[...].astype(BF),
            (((1,), (0,)), ((), ())), preferred_element_type=F32)

    out2 = pl.pallas_call(
        body,
        out_shape=jax.ShapeDtypeStruct((S, D), F32),
        in_specs=[pl.BlockSpec(memory_space=pltpu.MemorySpace.HBM)] * 5,
        out_specs=pl.BlockSpec(memory_space=pltpu.VMEM),
        scratch_shapes=[
            pltpu.VMEM((S, 2 * D), BF),
            pltpu.VMEM((3, 7, PMAX, 2 * D), BF),
            pltpu.VMEM((S, D), BF),
            pltpu.VMEM((S, D), F32),
            pltpu.VMEM((H, S, 1), BF),
            pltpu.VMEM((H, S, 1), F32),
            pltpu.VMEM((S, D), F32),
            pltpu.SemaphoreType.DMA((3, 7)),
            pltpu.SemaphoreType.DMA((3, 7)),
            pltpu.SemaphoreType.DMA((2,)),
        ],
        compiler_params=pltpu.CompilerParams(
            collective_id=0, vmem_limit_bytes=60 * 1024 * 1024),
    )(x2, Wq, k2, v2, Wo)
    return out2.reshape(1, S, D)


# device time: 244711 ns/iter; 1.4263x vs baseline; 1.0910x over previous
import jax
import jax.numpy as jnp
from jax import lax
from jax.experimental import pallas as pl
from jax.experimental.pallas import tpu as pltpu

N_DEV = 8
S = 1024
H = 8
DH = 128
D = H * DH
BLK = 64
QT = 512
SCALE = 0.08838834764831843
BF = jnp.bfloat16
F32 = jnp.float32

DIMS = (1, 3, 4)
ROW0 = (0, 344, 688)
NROWS = (344, 344, 336)
PMAX = 344


def _masks(a):
    return (DIMS[a], DIMS[(a + 1) % 3], DIMS[(a + 2) % 3])


def _gmask(a):
    m0, m1, m2 = _masks(a)
    return (m0, m1, m0 ^ m1, m2, m2 ^ m0, m2 ^ m1, m2 ^ m0 ^ m1)


_SCHED = {0: [(None, 0)],
          1: [(None, 1), (0, 2)],
          2: [(None, 3), (0, 4), (1, 5), (2, 6)]}


def kernel(x, Wq, K_ext, V_ext, Wo):
    x2 = x.reshape(S, D)
    k2 = K_ext.reshape(S, D)
    v2 = V_ext.reshape(S, D)

    def body(x_ref, wq_ref, k_ref, v_ref, wo_ref, out_ref,
             own, pieces, q_scr, acc, m_scr, l_scr, stg,
             send_sems, recv_sems, copy_sems):
        my = lax.axis_index("i")

        barrier = pltpu.get_barrier_semaphore()
        for mask in DIMS:
            pl.semaphore_signal(barrier, inc=1, device_id=(my ^ mask,),
                                device_id_type=pl.DeviceIdType.MESH)
        pl.semaphore_wait(barrier, 3)

        ck = pltpu.make_async_copy(k_ref, acc, copy_sems.at[0])
        cv = pltpu.make_async_copy(v_ref, stg, copy_sems.at[1])
        ck.start()
        cv.start()
        ck.wait()
        cv.wait()
        own[:, 0:D] = acc[...].astype(BF)
        own[:, D:2 * D] = stg[...].astype(BF)

        def mk(a, c, src_slot, dst):
            nr = NROWS[a]
            if src_slot is None:
                src = own.at[pl.ds(ROW0[a], nr), :]
            else:
                src = pieces.at[a, src_slot, pl.ds(0, nr), :]
            return pltpu.make_async_remote_copy(
                src_ref=src,
                dst_ref=pieces.at[a, c, pl.ds(0, nr), :],
                send_sem=send_sems.at[a, c],
                recv_sem=recv_sems.at[a, c],
                device_id=(dst,),
                device_id_type=pl.DeviceIdType.MESH)

        def do_round(a, r):
            partner = my ^ _masks(a)[r]
            descs = []
            for src_slot, c in _SCHED[r]:
                d = mk(a, c, src_slot, partner)
                d.start()
                descs.append(d)
            return descs

        def wait_round_recv(a, r):
            for src_slot, c in _SCHED[r]:
                mk(a, c, src_slot, 0).wait_recv()

        all_sends = []
        for a in range(3):
            all_sends += do_round(a, 0)

        cx = pltpu.make_async_copy(x_ref, acc, copy_sems.at[0])
        cw = pltpu.make_async_copy(wq_ref, stg, copy_sems.at[1])
        cx.start()
        cw.start()
        cx.wait()
        cw.wait()
        q = lax.dot_general(acc[...].astype(BF), stg[...].astype(BF),
                            (((1,), (0,)), ((), ())),
                            preferred_element_type=F32)
        q_scr[...] = (q * SCALE).astype(BF)

        m_scr[...] = jnp.full((H, S, 1), -1e30, BF)
        l_scr[...] = jnp.zeros((H, S, 1), F32)
        acc[...] = jnp.zeros((S, D), F32)

        def flash(kv_ref, o, row0, nrows, masked):
            @pl.when(o <= my)
            def _():
                def head(h, _):
                    kh = kv_ref[pl.ds(0, nrows) if kv_ref is not own
                                else pl.ds(row0, nrows), pl.ds(h * DH, DH)]
                    vh = kv_ref[pl.ds(0, nrows) if kv_ref is not own
                                else pl.ds(row0, nrows),
                                pl.ds(D + h * DH, DH)]

                    def tile(t, _):
                        r0q = t * QT
                        qh = q_scr[pl.ds(r0q, QT), pl.ds(h * DH, DH)]
                        s = lax.dot_general(qh, kh, (((1,), (1,)), ((), ())),
                                            preferred_element_type=F32)
                        if masked:
                            rows = lax.broadcasted_iota(
                                jnp.int32, (QT, nrows), 0) + r0q
                            cols = lax.broadcasted_iota(
                                jnp.int32, (QT, nrows), 1) + row0
                            bias = jnp.where(
                                cols // BLK <= rows // BLK, 0.0, -1e9
                            ).astype(F32)
                            s = s + bias
                        m_old = m_scr[h, pl.ds(r0q, QT)].astype(F32)
                        m_new = jnp.maximum(
                            m_old, jnp.max(s, axis=1, keepdims=True))
                        m_new = m_new.astype(BF).astype(F32)
                        p = jnp.exp(s - m_new)
                        alpha = jnp.exp(m_old - m_new)
                        l_scr[h, pl.ds(r0q, QT)] = (
                            l_scr[h, pl.ds(r0q, QT)] * alpha
                            + jnp.sum(p, axis=1, keepdims=True))
                        pv = lax.dot_general(p.astype(BF), vh,
                                             (((1,), (0,)), ((), ())),
                                             preferred_element_type=F32)
                        acc[pl.ds(r0q, QT), pl.ds(h * DH, DH)] = (
                            acc[pl.ds(r0q, QT), pl.ds(h * DH, DH)] * alpha
                            + pv)
                        m_scr[h, pl.ds(r0q, QT)] = m_new.astype(BF)
                        return 0

                    lax.fori_loop(0, S // QT, tile, 0)
                    return 0

                lax.fori_loop(0, H, head, 0)

        def process_piece(a, c):
            flash(pieces.at[a, c], my ^ _gmask(a)[c], ROW0[a], NROWS[a],
                  masked=False)

        flash(own, my, 0, S, masked=True)

        for a in range(3):
            wait_round_recv(a, 0)
            all_sends += do_round(a, 1)
        for a in range(3):
            process_piece(a, 0)

        for a in range(3):
            wait_round_recv(a, 1)
            all_sends += do_round(a, 2)
        cwo = pltpu.make_async_copy(wo_ref, stg, copy_sems.at[0])
        cwo.start()
        for a in range(3):
            process_piece(a, 1)
            process_piece(a, 2)

        for a in range(3):
            wait_round_recv(a, 2)
        for a in range(3):
            for c in (3, 4, 5, 6):
                process_piece(a, c)

        for dsc in all_sends:
            dsc.wait_send()

        def fin(h, _):
            acc[:, pl.ds(h * DH, DH)] = acc[:, pl.ds(h * DH, DH)] / l_scr[h]
            return 0
        lax.fori_loop(0, H, fin, 0)

        cwo.wait()
        out_ref[...] = lax.dot_general(
            acc[...].astype(BF), stg[...].astype(BF),
            (((1,), (0,)), ((), ())), preferred_element_type=F32)

    out2 = pl.pallas_call(
        body,
        out_shape=jax.ShapeDtypeStruct((S, D), F32),
        in_specs=[pl.BlockSpec(memory_space=pltpu.MemorySpace.HBM)] * 5,
        out_specs=pl.BlockSpec(memory_space=pltpu.VMEM),
        scratch_shapes=[
            pltpu.VMEM((S, 2 * D), BF),
            pltpu.VMEM((3, 7, PMAX, 2 * D), BF),
            pltpu.VMEM((S, D), BF),
            pltpu.VMEM((S, D), F32),
            pltpu.VMEM((H, S, 1), BF),
            pltpu.VMEM((H, S, 1), F32),
            pltpu.VMEM((S, D), F32),
            pltpu.SemaphoreType.DMA((3, 7)),
            pltpu.SemaphoreType.DMA((3, 7)),
            pltpu.SemaphoreType.DMA((2,)),
        ],
        compiler_params=pltpu.CompilerParams(
            collective_id=0, vmem_limit_bytes=60 * 1024 * 1024),
    )(x2, Wq, k2, v2, Wo)
    return out2.reshape(1, S, D)


# device time: 219217 ns/iter; 1.5921x vs baseline; 1.1163x over previous
import jax
import jax.numpy as jnp
from jax import lax
from jax.experimental import pallas as pl
from jax.experimental.pallas import tpu as pltpu

N_DEV = 8
S = 1024
H = 8
DH = 128
D = H * DH
BLK = 64
QT = 512
SCALE = 0.08838834764831843
BF = jnp.bfloat16
F32 = jnp.float32

DIMS = (1, 3, 4)
ROW0 = (0, 344, 688)
NROWS = (344, 344, 336)
PMAX = 344


def _masks(a):
    return (DIMS[a], DIMS[(a + 1) % 3], DIMS[(a + 2) % 3])


def _gmask(a):
    m0, m1, m2 = _masks(a)
    return (m0, m1, m0 ^ m1, m2, m2 ^ m0, m2 ^ m1, m2 ^ m0 ^ m1)


_SCHED = {0: [(None, 0)],
          1: [(None, 1), (0, 2)],
          2: [(None, 3), (0, 4), (1, 5), (2, 6)]}


def kernel(x, Wq, K_ext, V_ext, Wo):
    x2 = x.reshape(S, D)
    k2 = K_ext.reshape(S, D)
    v2 = V_ext.reshape(S, D)

    def body(x_ref, wq_ref, k_ref, v_ref, wo_ref, out_ref,
             own, pieces, q_scr, acc, m_scr, l_scr, stg,
             send_sems, recv_sems, copy_sems):
        my = lax.axis_index("i")

        barrier = pltpu.get_barrier_semaphore()
        for mask in DIMS:
            pl.semaphore_signal(barrier, inc=1, device_id=(my ^ mask,),
                                device_id_type=pl.DeviceIdType.MESH)
        pl.semaphore_wait(barrier, 3)

        ck = pltpu.make_async_copy(k_ref, acc, copy_sems.at[0])
        cv = pltpu.make_async_copy(v_ref, stg, copy_sems.at[1])
        ck.start()
        cv.start()
        ck.wait()
        cv.wait()
        own[:, 0:D] = acc[...].astype(BF)
        own[:, D:2 * D] = stg[...].astype(BF)

        def mk(a, c, src_slot, dst):
            nr = NROWS[a]
            if src_slot is None:
                src = own.at[pl.ds(ROW0[a], nr), :]
            else:
                src = pieces.at[a, src_slot, pl.ds(0, nr), :]
            return pltpu.make_async_remote_copy(
                src_ref=src,
                dst_ref=pieces.at[a, c, pl.ds(0, nr), :],
                send_sem=send_sems.at[a, c],
                recv_sem=recv_sems.at[a, c],
                device_id=(dst,),
                device_id_type=pl.DeviceIdType.MESH)

        def do_round(a, r):
            partner = my ^ _masks(a)[r]
            descs = []
            for src_slot, c in _SCHED[r]:
                d = mk(a, c, src_slot, partner)
                d.start()
                descs.append(d)
            return descs

        def wait_round_recv(a, r):
            for src_slot, c in _SCHED[r]:
                mk(a, c, src_slot, 0).wait_recv()

        all_sends = []
        for a in range(3):
            all_sends += do_round(a, 0)

        cx = pltpu.make_async_copy(x_ref, acc, copy_sems.at[0])
        cw = pltpu.make_async_copy(wq_ref, stg, copy_sems.at[1])
        cx.start()
        cw.start()
        cx.wait()
        cw.wait()
        q = lax.dot_general(acc[...].astype(BF), stg[...].astype(BF),
                            (((1,), (0,)), ((), ())),
                            preferred_element_type=F32)
        q_scr[...] = (q * SCALE).astype(BF)

        m_scr[...] = jnp.full((H, S, 1), -1e30, BF)
        l_scr[...] = jnp.zeros((H, S, 1), F32)
        acc[...] = jnp.zeros((S, D), F32)

        def flash(kv_ref, o, row0, nrows, masked):
            @pl.when(o <= my)
            def _():
                def head(h, _):
                    kh = kv_ref[pl.ds(0, nrows) if kv_ref is not own
                                else pl.ds(row0, nrows), pl.ds(h * DH, DH)]
                    vh = kv_ref[pl.ds(0, nrows) if kv_ref is not own
                                else pl.ds(row0, nrows),
                                pl.ds(D + h * DH, DH)]

                    def tile(t, _):
                        r0q = t * QT
                        qh = q_scr[pl.ds(r0q, QT), pl.ds(h * DH, DH)]
                        s = lax.dot_general(qh, kh, (((1,), (1,)), ((), ())),
                                            preferred_element_type=F32)
                        if masked:
                            rows = lax.broadcasted_iota(
                                jnp.int32, (QT, nrows), 0) + r0q
                            cols = lax.broadcasted_iota(
                                jnp.int32, (QT, nrows), 1) + row0
                            bias = jnp.where(
                                cols // BLK <= rows // BLK, 0.0, -1e9
                            ).astype(F32)
                            s = s + bias
                        m_old = m_scr[h, pl.ds(r0q, QT)].astype(F32)
                        m_new = jnp.maximum(
                            m_old, jnp.max(s, axis=1, keepdims=True))
                        m_new = m_new.astype(BF).astype(F32)
                        p = jnp.exp(s - m_new)
                        alpha = jnp.exp(m_old - m_new)
                        l_scr[h, pl.ds(r0q, QT)] = (
                            l_scr[h, pl.ds(r0q, QT)] * alpha
                            + jnp.sum(p, axis=1, keepdims=True))
                        pv = lax.dot_general(p.astype(BF), vh,
                                             (((1,), (0,)), ((), ())),
                                             preferred_element_type=F32)
                        acc[pl.ds(r0q, QT), pl.ds(h * DH, DH)] = (
                            acc[pl.ds(r0q, QT), pl.ds(h * DH, DH)] * alpha
                            + pv)
                        m_scr[h, pl.ds(r0q, QT)] = m_new.astype(BF)
                        return 0

                    lax.fori_loop(0, S // QT, tile, 0)
                    return 0

                lax.fori_loop(0, H, head, 0)

        def process_piece(a, c):
            flash(pieces.at[a, c], my ^ _gmask(a)[c], ROW0[a], NROWS[a],
                  masked=False)

        flash(own, my, 0, S, masked=True)

        for a in range(3):
            wait_round_recv(a, 0)
            all_sends += do_round(a, 1)
        for a in range(3):
            process_piece(a, 0)

        for a in range(3):
            wait_round_recv(a, 1)
            all_sends += do_round(a, 2)
        cwo = pltpu.make_async_copy(wo_ref, stg, copy_sems.at[0])
        cwo.start()
        for a in range(3):
            process_piece(a, 1)
            process_piece(a, 2)

        for a in range(3):
            for src_slot, c in _SCHED[2][:2]:
                mk(a, c, src_slot, 0).wait_recv()
        for a in range(3):
            process_piece(a, 3)
            process_piece(a, 4)
        for a in range(3):
            for src_slot, c in _SCHED[2][2:]:
                mk(a, c, src_slot, 0).wait_recv()
        for a in range(3):
            process_piece(a, 5)
            process_piece(a, 6)

        for dsc in all_sends:
            dsc.wait_send()

        def fin(h, _):
            acc[:, pl.ds(h * DH, DH)] = acc[:, pl.ds(h * DH, DH)] / l_scr[h]
            return 0
        lax.fori_loop(0, H, fin, 0)

        cwo.wait()
        out_ref[...] = lax.dot_general(
            acc[...].astype(BF), stg[...].astype(BF),
            (((1,), (0,)), ((), ())), preferred_element_type=F32)

    out2 = pl.pallas_call(
        body,
        out_shape=jax.ShapeDtypeStruct((S, D), F32),
        in_specs=[pl.BlockSpec(memory_space=pltpu.MemorySpace.HBM)] * 5,
        out_specs=pl.BlockSpec(memory_space=pltpu.VMEM),
        scratch_shapes=[
            pltpu.VMEM((S, 2 * D), BF),
            pltpu.VMEM((3, 7, PMAX, 2 * D), BF),
            pltpu.VMEM((S, D), BF),
            pltpu.VMEM((S, D), F32),
            pltpu.VMEM((H, S, 1), BF),
            pltpu.VMEM((H, S, 1), F32),
            pltpu.VMEM((S, D), F32),
            pltpu.SemaphoreType.DMA((3, 7)),
            pltpu.SemaphoreType.DMA((3, 7)),
            pltpu.SemaphoreType.DMA((2,)),
        ],
        compiler_params=pltpu.CompilerParams(
            collective_id=0, vmem_limit_bytes=60 * 1024 * 1024),
    )(x2, Wq, k2, v2, Wo)
    return out2.reshape(1, S, D)


# device time: 216181 ns/iter; 1.6145x vs baseline; 1.0140x over previous
import jax
import jax.numpy as jnp
from jax import lax
from jax.experimental import pallas as pl
from jax.experimental.pallas import tpu as pltpu

N_DEV = 8
S = 1024
H = 8
DH = 128
D = H * DH
BLK = 64
QT = 512
SCALE = 0.08838834764831843
BF = jnp.bfloat16
F32 = jnp.float32

DIMS = (1, 3, 4)
ROW0 = (0, 344, 688)
NROWS = (344, 344, 336)
PMAX = 344


def _masks(a):
    return (DIMS[a], DIMS[(a + 1) % 3], DIMS[(a + 2) % 3])


def _gmask(a):
    m0, m1, m2 = _masks(a)
    return (m0, m1, m0 ^ m1, m2, m2 ^ m0, m2 ^ m1, m2 ^ m0 ^ m1)


_SCHED = {0: [(None, 0)],
          1: [(None, 1), (0, 2)],
          2: [(None, 3), (0, 4), (1, 5), (2, 6)]}


def kernel(x, Wq, K_ext, V_ext, Wo):
    x2 = x.reshape(S, D)
    k2 = K_ext.reshape(S, D)
    v2 = V_ext.reshape(S, D)

    def body(x_ref, wq_ref, k_ref, v_ref, wo_ref, out_ref,
             own, pieces, q_scr, acc, m_scr, l_scr, stg,
             send_sems, recv_sems, copy_sems):
        my = lax.axis_index("i")

        barrier = pltpu.get_barrier_semaphore()
        for mask in DIMS:
            pl.semaphore_signal(barrier, inc=1, device_id=(my ^ mask,),
                                device_id_type=pl.DeviceIdType.MESH)
        pl.semaphore_wait(barrier, 3)

        ck = pltpu.make_async_copy(k_ref, acc, copy_sems.at[0])
        cv = pltpu.make_async_copy(v_ref, stg, copy_sems.at[1])
        ck.start()
        cv.start()
        ck.wait()
        cv.wait()
        own[:, 0:D] = acc[...].astype(BF)
        own[:, D:2 * D] = stg[...].astype(BF)

        def mk(a, c, src_slot, dst):
            nr = NROWS[a]
            if src_slot is None:
                src = own.at[pl.ds(ROW0[a], nr), :]
            else:
                src = pieces.at[a, src_slot, pl.ds(0, nr), :]
            return pltpu.make_async_remote_copy(
                src_ref=src,
                dst_ref=pieces.at[a, c, pl.ds(0, nr), :],
                send_sem=send_sems.at[a, c],
                recv_sem=recv_sems.at[a, c],
                device_id=(dst,),
                device_id_type=pl.DeviceIdType.MESH)

        def do_round(a, r):
            partner = my ^ _masks(a)[r]
            descs = []
            for src_slot, c in _SCHED[r]:
                d = mk(a, c, src_slot, partner)
                d.start()
                descs.append(d)
            return descs

        def wait_round_recv(a, r):
            for src_slot, c in _SCHED[r]:
                mk(a, c, src_slot, 0).wait_recv()

        all_sends = []
        for a in range(3):
            all_sends += do_round(a, 0)

        cx = pltpu.make_async_copy(x_ref, acc, copy_sems.at[0])
        cw = pltpu.make_async_copy(wq_ref, stg, copy_sems.at[1])
        cx.start()
        cw.start()
        cx.wait()
        cw.wait()
        q = lax.dot_general(acc[...].astype(BF), stg[...].astype(BF),
                            (((1,), (0,)), ((), ())),
                            preferred_element_type=F32)
        q_scr[...] = (q * SCALE).astype(BF)

        m_scr[...] = jnp.full((H, S, 1), -1e30, BF)
        l_scr[...] = jnp.zeros((H, S, 1), F32)
        acc[...] = jnp.zeros((S, D), F32)

        def flash(kv_ref, o, row0, nrows, masked, qt=1024):
            @pl.when(o <= my)
            def _():
                def head(h, _):
                    kh = kv_ref[pl.ds(0, nrows) if kv_ref is not own
                                else pl.ds(row0, nrows), pl.ds(h * DH, DH)]
                    vh = kv_ref[pl.ds(0, nrows) if kv_ref is not own
                                else pl.ds(row0, nrows),
                                pl.ds(D + h * DH, DH)]

                    def tile(t, _):
                        r0q = t * qt
                        qh = q_scr[pl.ds(r0q, qt), pl.ds(h * DH, DH)]
                        s = lax.dot_general(qh, kh, (((1,), (1,)), ((), ())),
                                            preferred_element_type=F32)
                        if masked:
                            rows = lax.broadcasted_iota(
                                jnp.int32, (qt, nrows), 0) + r0q
                            cols = lax.broadcasted_iota(
                                jnp.int32, (qt, nrows), 1) + row0
                            bias = jnp.where(
                                cols // BLK <= rows // BLK, 0.0, -1e9
                            ).astype(F32)
                            s = s + bias
                        m_old = m_scr[h, pl.ds(r0q, qt)].astype(F32)
                        m_new = jnp.maximum(
                            m_old, jnp.max(s, axis=1, keepdims=True))
                        m_new = m_new.astype(BF).astype(F32)
                        p = jnp.exp(s - m_new)
                        alpha = jnp.exp(m_old - m_new)
                        l_scr[h, pl.ds(r0q, qt)] = (
                            l_scr[h, pl.ds(r0q, qt)] * alpha
                            + jnp.sum(p, axis=1, keepdims=True))
                        pv = lax.dot_general(p.astype(BF), vh,
                                             (((1,), (0,)), ((), ())),
                                             preferred_element_type=F32)
                        acc[pl.ds(r0q, qt), pl.ds(h * DH, DH)] = (
                            acc[pl.ds(r0q, qt), pl.ds(h * DH, DH)] * alpha
                            + pv)
                        m_scr[h, pl.ds(r0q, qt)] = m_new.astype(BF)
                        return 0

                    lax.fori_loop(0, S // qt, tile, 0)
                    return 0

                lax.fori_loop(0, H, head, 0)

        def process_piece(a, c):
            flash(pieces.at[a, c], my ^ _gmask(a)[c], ROW0[a], NROWS[a],
                  masked=False)

        flash(own, my, 0, S, masked=True, qt=QT)

        for a in range(3):
            wait_round_recv(a, 0)
            all_sends += do_round(a, 1)
        for a in range(3):
            process_piece(a, 0)

        for a in range(3):
            wait_round_recv(a, 1)
            all_sends += do_round(a, 2)
        cwo = pltpu.make_async_copy(wo_ref, stg, copy_sems.at[0])
        cwo.start()
        for a in range(3):
            process_piece(a, 1)
            process_piece(a, 2)

        for a in range(3):
            for src_slot, c in _SCHED[2][:2]:
                mk(a, c, src_slot, 0).wait_recv()
        for a in range(3):
            process_piece(a, 3)
            process_piece(a, 4)
        for a in range(3):
            for src_slot, c in _SCHED[2][2:]:
                mk(a, c, src_slot, 0).wait_recv()
        for a in range(3):
            process_piece(a, 5)
            process_piece(a, 6)

        for dsc in all_sends:
            dsc.wait_send()

        def fin(h, _):
            acc[:, pl.ds(h * DH, DH)] = acc[:, pl.ds(h * DH, DH)] / l_scr[h]
            return 0
        lax.fori_loop(0, H, fin, 0)

        cwo.wait()
        out_ref[...] = lax.dot_general(
            acc[...].astype(BF), stg[...].astype(BF),
            (((1,), (0,)), ((), ())), preferred_element_type=F32)

    out2 = pl.pallas_call(
        body,
        out_shape=jax.ShapeDtypeStruct((S, D), F32),
        in_specs=[pl.BlockSpec(memory_space=pltpu.MemorySpace.HBM)] * 5,
        out_specs=pl.BlockSpec(memory_space=pltpu.VMEM),
        scratch_shapes=[
            pltpu.VMEM((S, 2 * D), BF),
            pltpu.VMEM((3, 7, PMAX, 2 * D), BF),
            pltpu.VMEM((S, D), BF),
            pltpu.VMEM((S, D), F32),
            pltpu.VMEM((H, S, 1), BF),
            pltpu.VMEM((H, S, 1), F32),
            pltpu.VMEM((S, D), F32),
            pltpu.SemaphoreType.DMA((3, 7)),
            pltpu.SemaphoreType.DMA((3, 7)),
            pltpu.SemaphoreType.DMA((2,)),
        ],
        compiler_params=pltpu.CompilerParams(
            collective_id=0, vmem_limit_bytes=60 * 1024 * 1024),
    )(x2, Wq, k2, v2, Wo)
    return out2.reshape(1, S, D)


# device time: 216025 ns/iter; 1.6156x vs baseline; 1.0007x over previous
import jax
import jax.numpy as jnp
from jax import lax
from jax.experimental import pallas as pl
from jax.experimental.pallas import tpu as pltpu

N_DEV = 8
S = 1024
H = 8
DH = 128
D = H * DH
BLK = 64
QT = 512
SCALE = 0.08838834764831843
BF = jnp.bfloat16
F32 = jnp.float32

DIMS = (1, 3, 4)
ROW0 = (0, 344, 688)
NROWS = (344, 344, 336)
PMAX = 344


def _masks(a):
    return (DIMS[a], DIMS[(a + 1) % 3], DIMS[(a + 2) % 3])


def _gmask(a):
    m0, m1, m2 = _masks(a)
    return (m0, m1, m0 ^ m1, m2, m2 ^ m0, m2 ^ m1, m2 ^ m0 ^ m1)


_SCHED = {0: [(None, 0)],
          1: [(None, 1), (0, 2)],
          2: [(None, 3), (0, 4), (1, 5), (2, 6)]}


def kernel(x, Wq, K_ext, V_ext, Wo):
    x2 = x.reshape(S, D)
    k2 = K_ext.reshape(S, D)
    v2 = V_ext.reshape(S, D)

    def body(x_ref, wq_ref, k_ref, v_ref, wo_ref, out_ref,
             own, pieces, q_scr, acc, m_scr, l_scr, stg,
             send_sems, recv_sems, copy_sems):
        my = lax.axis_index("i")

        barrier = pltpu.get_barrier_semaphore()
        for mask in DIMS:
            pl.semaphore_signal(barrier, inc=1, device_id=(my ^ mask,),
                                device_id_type=pl.DeviceIdType.MESH)
        pl.semaphore_wait(barrier, 3)

        ck = pltpu.make_async_copy(k_ref, acc, copy_sems.at[0])
        cv = pltpu.make_async_copy(v_ref, stg, copy_sems.at[1])
        ck.start()
        cv.start()
        ck.wait()
        cv.wait()
        own[:, 0:D] = acc[...].astype(BF)
        own[:, D:2 * D] = stg[...].astype(BF)

        def mk(a, c, src_slot, dst):
            nr = NROWS[a]
            if src_slot is None:
                src = own.at[pl.ds(ROW0[a], nr), :]
            else:
                src = pieces.at[a, src_slot, pl.ds(0, nr), :]
            return pltpu.make_async_remote_copy(
                src_ref=src,
                dst_ref=pieces.at[a, c, pl.ds(0, nr), :],
                send_sem=send_sems.at[a, c],
                recv_sem=recv_sems.at[a, c],
                device_id=(dst,),
                device_id_type=pl.DeviceIdType.MESH)

        def do_round(a, r):
            partner = my ^ _masks(a)[r]
            descs = []
            for src_slot, c in _SCHED[r]:
                d = mk(a, c, src_slot, partner)
                d.start()
                descs.append(d)
            return descs

        def wait_round_recv(a, r):
            for src_slot, c in _SCHED[r]:
                mk(a, c, src_slot, 0).wait_recv()

        all_sends = []
        for a in range(3):
            all_sends += do_round(a, 0)

        cx = pltpu.make_async_copy(x_ref, acc, copy_sems.at[0])
        cw = pltpu.make_async_copy(wq_ref, stg, copy_sems.at[1])
        cx.start()
        cw.start()
        cx.wait()
        cw.wait()
        q = lax.dot_general(acc[...].astype(BF), stg[...].astype(BF),
                            (((1,), (0,)), ((), ())),
                            preferred_element_type=F32)
        q_scr[...] = (q * SCALE).astype(BF)

        m_scr[...] = jnp.full((H, S, 1), -1e30, BF)
        l_scr[...] = jnp.zeros((H, S, 1), F32)
        acc[...] = jnp.zeros((S, D), F32)

        def flash(kv_ref, o, row0, nrows, masked, qt=1024):
            @pl.when(o <= my)
            def _():
                def head(h, _):
                    kh = kv_ref[pl.ds(0, nrows) if kv_ref is not own
                                else pl.ds(row0, nrows), pl.ds(h * DH, DH)]
                    vh = kv_ref[pl.ds(0, nrows) if kv_ref is not own
                                else pl.ds(row0, nrows),
                                pl.ds(D + h * DH, DH)]

                    def tile(t, _):
                        r0q = t * qt
                        qh = q_scr[pl.ds(r0q, qt), pl.ds(h * DH, DH)]
                        s = lax.dot_general(qh, kh, (((1,), (1,)), ((), ())),
                                            preferred_element_type=F32)
                        if masked:
                            rows = lax.broadcasted_iota(
                                jnp.int32, (qt, nrows), 0) + r0q
                            cols = lax.broadcasted_iota(
                                jnp.int32, (qt, nrows), 1) + row0
                            bias = jnp.where(
                                cols // BLK <= rows // BLK, 0.0, -1e9
                            ).astype(F32)
                            s = s + bias
                        m_old = m_scr[h, pl.ds(r0q, qt)].astype(F32)
                        m_new = jnp.maximum(
                            m_old, jnp.max(s, axis=1, keepdims=True))
                        m_new = m_new.astype(BF).astype(F32)
                        p = jnp.exp((s - m_new).astype(BF))
                        alpha = jnp.exp(m_old - m_new)
                        l_scr[h, pl.ds(r0q, qt)] = (
                            l_scr[h, pl.ds(r0q, qt)] * alpha
                            + jnp.sum(p, axis=1, keepdims=True,
                                      dtype=jnp.float32))
                        pv = lax.dot_general(p, vh,
                                             (((1,), (0,)), ((), ())),
                                             preferred_element_type=F32)
                        acc[pl.ds(r0q, qt), pl.ds(h * DH, DH)] = (
                            acc[pl.ds(r0q, qt), pl.ds(h * DH, DH)] * alpha
                            + pv)
                        m_scr[h, pl.ds(r0q, qt)] = m_new.astype(BF)
                        return 0

                    lax.fori_loop(0, S // qt, tile, 0)
                    return 0

                lax.fori_loop(0, H, head, 0)

        def process_piece(a, c):
            flash(pieces.at[a, c], my ^ _gmask(a)[c], ROW0[a], NROWS[a],
                  masked=False)

        flash(own, my, 0, S, masked=True, qt=QT)

        for a in range(3):
            wait_round_recv(a, 0)
            all_sends += do_round(a, 1)
        for a in range(3):
            process_piece(a, 0)

        for a in range(3):
            wait_round_recv(a, 1)
            all_sends += do_round(a, 2)
        cwo = pltpu.make_async_copy(wo_ref, stg, copy_sems.at[0])
        cwo.start()
        for a in range(3):
            process_piece(a, 1)
            process_piece(a, 2)

        for a in range(3):
            for src_slot, c in _SCHED[2][:2]:
                mk(a, c, src_slot, 0).wait_recv()
        for a in range(3):
            process_piece(a, 3)
            process_piece(a, 4)
        for a in range(3):
            for src_slot, c in _SCHED[2][2:]:
                mk(a, c, src_slot, 0).wait_recv()
        for a in range(3):
            process_piece(a, 5)
            process_piece(a, 6)

        for dsc in all_sends:
            dsc.wait_send()

        def fin(h, _):
            acc[:, pl.ds(h * DH, DH)] = acc[:, pl.ds(h * DH, DH)] / l_scr[h]
            return 0
        lax.fori_loop(0, H, fin, 0)

        cwo.wait()
        out_ref[...] = lax.dot_general(
            acc[...].astype(BF), stg[...].astype(BF),
            (((1,), (0,)), ((), ())), preferred_element_type=F32)

    out2 = pl.pallas_call(
        body,
        out_shape=jax.ShapeDtypeStruct((S, D), F32),
        in_specs=[pl.BlockSpec(memory_space=pltpu.MemorySpace.HBM)] * 5,
        out_specs=pl.BlockSpec(memory_space=pltpu.VMEM),
        scratch_shapes=[
            pltpu.VMEM((S, 2 * D), BF),
            pltpu.VMEM((3, 7, PMAX, 2 * D), BF),
            pltpu.VMEM((S, D), BF),
            pltpu.VMEM((S, D), F32),
            pltpu.VMEM((H, S, 1), BF),
            pltpu.VMEM((H, S, 1), F32),
            pltpu.VMEM((S, D), F32),
            pltpu.SemaphoreType.DMA((3, 7)),
            pltpu.SemaphoreType.DMA((3, 7)),
            pltpu.SemaphoreType.DMA((2,)),
        ],
        compiler_params=pltpu.CompilerParams(
            collective_id=0, vmem_limit_bytes=60 * 1024 * 1024),
    )(x2, Wq, k2, v2, Wo)
    return out2.reshape(1, S, D)
